# Initial kernel scaffold; baseline (speedup 1.0000x reference)
import jax
import jax.numpy as jnp
from jax.experimental import pallas as pl


def kernel(x, edge_index, W1, b1, W2, b2):
    raise NotImplementedError("write your pallas kernel here")


# speedup vs baseline: 9.1275x; 9.1275x over previous
import functools

import jax
import jax.numpy as jnp
from jax import lax
from jax.experimental import pallas as pl
from jax.experimental.pallas import tpu as pltpu
from jax.experimental.pallas import tpu_sc as plsc

N = 10000
D = 128
NPAD = 10240
NC = 2
NS = 16
NW = NC * NS
CHUNK = 128
R_EDGE = 2560
R_W = R_EDGE // NW
STRIPE = NPAD // NS
BR = 512

_mesh = plsc.VectorSubcoreMesh(core_axis_name="c", subcore_axis_name="s")


@functools.partial(
    pl.kernel,
    out_type=jax.ShapeDtypeStruct((NC, NPAD, D), jnp.float32),
    mesh=_mesh,
    scratch_types=[
        pltpu.VMEM((R_W, CHUNK), jnp.int32),
        pltpu.VMEM((CHUNK, D), jnp.float32),
        pltpu.VMEM_SHARED((NPAD, D), jnp.float32),
    ],
)
def _sc_hist(dstp_hbm, zeros_hbm, ones_hbm, out, dst_v, ones_v, hist_sh):
    c = lax.axis_index("c")
    s = lax.axis_index("s")
    w = c * NS + s
    pltpu.sync_copy(dstp_hbm.at[pl.ds(w * R_W, R_W)], dst_v)
    pltpu.sync_copy(ones_hbm, ones_v)
    pltpu.sync_copy(zeros_hbm, hist_sh.at[pl.ds(s * STRIPE, STRIPE)])
    plsc.subcore_barrier()

    def body(j, carry):
        pltpu.sync_copy(ones_v, hist_sh.at[dst_v.at[j]], add=True)
        return carry

    lax.fori_loop(0, R_W, body, 0)
    plsc.subcore_barrier()
    pltpu.sync_copy(hist_sh.at[pl.ds(s * STRIPE, STRIPE)],
                    out.at[c, pl.ds(s * STRIPE, STRIPE)])


@functools.partial(
    pl.kernel,
    out_type=jax.ShapeDtypeStruct((NC, NPAD, D), jnp.float32),
    mesh=_mesh,
    scratch_types=[
        pltpu.VMEM((R_W, CHUNK), jnp.int32),
        pltpu.VMEM((R_W, CHUNK), jnp.int32),
        pltpu.VMEM((CHUNK, D), jnp.float32),
        pltpu.VMEM_SHARED((NPAD, D), jnp.float32),
        pltpu.SemaphoreType.DMA,
    ],
)
def _sc_scatter(g_hbm, srcp_hbm, dstp_hbm, zeros_hbm, out,
                src_v, dst_v, rows_v, acc_sh, sem):
    c = lax.axis_index("c")
    s = lax.axis_index("s")
    w = c * NS + s
    pltpu.sync_copy(srcp_hbm.at[pl.ds(w * R_W, R_W)], src_v)
    pltpu.sync_copy(dstp_hbm.at[pl.ds(w * R_W, R_W)], dst_v)
    pltpu.sync_copy(zeros_hbm, acc_sh.at[pl.ds(s * STRIPE, STRIPE)])
    plsc.subcore_barrier()

    def body(j, carry):
        pltpu.async_copy(g_hbm.at[src_v.at[j]], rows_v, sem).wait()
        pltpu.sync_copy(rows_v, acc_sh.at[dst_v.at[j]], add=True)
        return carry

    lax.fori_loop(0, R_W, body, 0)
    plsc.subcore_barrier()
    pltpu.sync_copy(acc_sh.at[pl.ds(s * STRIPE, STRIPE)],
                    out.at[c, pl.ds(s * STRIPE, STRIPE)])


def _mm_body(x_ref, w_ref, o_ref):
    o_ref[...] = jnp.dot(x_ref[...], w_ref[...],
                         preferred_element_type=jnp.float32)


def _tc_matmul(xp, W):
    return pl.pallas_call(
        _mm_body,
        grid=(NPAD // BR,),
        in_specs=[pl.BlockSpec((BR, D), lambda i: (i, 0)),
                  pl.BlockSpec((D, D), lambda i: (0, 0))],
        out_specs=pl.BlockSpec((BR, D), lambda i: (i, 0)),
        out_shape=jax.ShapeDtypeStruct((NPAD, D), jnp.float32),
    )(xp, W)


_HSPEC0 = pl.BlockSpec((1, BR, D), lambda i: (0, i, 0))
_HSPEC1 = pl.BlockSpec((1, BR, D), lambda i: (1, i, 0))
_PSPEC0 = pl.BlockSpec((1, BR, D), lambda i: (0, i, 0))
_PSPEC1 = pl.BlockSpec((1, BR, D), lambda i: (1, i, 0))


def _dinv_of(h0_ref, h1_ref):
    deg = h0_ref[0] + h1_ref[0] + 1.0
    return lax.rsqrt(deg)


def _scale_body(h_ref, h0_ref, h1_ref, o_ref):
    o_ref[...] = h_ref[...] * _dinv_of(h0_ref, h1_ref)


def _tc_scale(H, hp):
    return pl.pallas_call(
        _scale_body,
        grid=(NPAD // BR,),
        in_specs=[pl.BlockSpec((BR, D), lambda i: (i, 0)), _HSPEC0, _HSPEC1],
        out_specs=pl.BlockSpec((BR, D), lambda i: (i, 0)),
        out_shape=jax.ShapeDtypeStruct((NPAD, D), jnp.float32),
    )(H, hp, hp)


def _layer_body(p_ref, q_ref, g_ref, h0_ref, h1_ref, b_ref, w_ref, o_ref):
    dinv = _dinv_of(h0_ref, h1_ref)
    hmid = jnp.maximum(
        dinv * (p_ref[0] + q_ref[0] + g_ref[...]) + b_ref[...], 0.0)
    o_ref[...] = dinv * jnp.dot(hmid, w_ref[...],
                                preferred_element_type=jnp.float32)


def _tc_layer(p, g, hp, b, W):
    return pl.pallas_call(
        _layer_body,
        grid=(NPAD // BR,),
        in_specs=[_PSPEC0, _PSPEC1,
                  pl.BlockSpec((BR, D), lambda i: (i, 0)),
                  _HSPEC0, _HSPEC1,
                  pl.BlockSpec((1, D), lambda i: (0, 0)),
                  pl.BlockSpec((D, D), lambda i: (0, 0))],
        out_specs=pl.BlockSpec((BR, D), lambda i: (i, 0)),
        out_shape=jax.ShapeDtypeStruct((NPAD, D), jnp.float32),
    )(p, p, g, hp, hp, b, W)


def _final_body(p_ref, q_ref, g_ref, h0_ref, h1_ref, b_ref, o_ref):
    dinv = _dinv_of(h0_ref, h1_ref)
    o_ref[...] = dinv * (p_ref[0] + q_ref[0] + g_ref[...]) + b_ref[...]


def _tc_final(p, g, hp, b):
    return pl.pallas_call(
        _final_body,
        grid=(NPAD // BR,),
        in_specs=[_PSPEC0, _PSPEC1,
                  pl.BlockSpec((BR, D), lambda i: (i, 0)),
                  _HSPEC0, _HSPEC1,
                  pl.BlockSpec((1, D), lambda i: (0, 0))],
        out_specs=pl.BlockSpec((BR, D), lambda i: (i, 0)),
        out_shape=jax.ShapeDtypeStruct((NPAD, D), jnp.float32),
    )(p, p, g, hp, hp, b)


def kernel(x, edge_index, W1, b1, W2, b2):
    src = edge_index[0]
    dst = edge_index[1]
    e = src.shape[0]
    fill = jnp.full((R_EDGE * CHUNK - e,), N, dtype=jnp.int32)
    srcp = jnp.concatenate([src, fill]).reshape(R_EDGE, CHUNK)
    dstp = jnp.concatenate([dst, fill]).reshape(R_EDGE, CHUNK)
    xp = jnp.pad(x, ((0, NPAD - N), (0, 0)))
    zD = jnp.zeros((STRIPE, D), jnp.float32)
    onesD = jnp.ones((CHUNK, D), jnp.float32)

    hp = _sc_hist(dstp, zD, onesD)
    H = _tc_matmul(xp, W1)
    g1 = _tc_scale(H, hp)
    p = _sc_scatter(g1, srcp, dstp, zD)
    g2 = _tc_layer(p, g1, hp, b1.reshape(1, D), W2)
    q = _sc_scatter(g2, srcp, dstp, zD)
    outp = _tc_final(q, g2, hp, b2.reshape(1, D))
    return outp[:N]


# speedup vs baseline: 9.8979x; 1.0844x over previous
import functools

import jax
import jax.numpy as jnp
from jax import lax
from jax.experimental import pallas as pl
from jax.experimental.pallas import tpu as pltpu
from jax.experimental.pallas import tpu_sc as plsc

N = 10000
D = 128
NPAD = 10240
NC = 2
NS = 16
NW = NC * NS
CHUNK = 128
R_EDGE = 2560
R_W = R_EDGE // NW
STRIPE = NPAD // NS
BR = 512

_mesh = plsc.VectorSubcoreMesh(core_axis_name="c", subcore_axis_name="s")


@functools.partial(
    pl.kernel,
    out_type=jax.ShapeDtypeStruct((NC, NPAD, D), jnp.float32),
    mesh=_mesh,
    scratch_types=[
        pltpu.VMEM((R_W, CHUNK), jnp.int32),
        pltpu.VMEM((CHUNK, D), jnp.float32),
        pltpu.VMEM_SHARED((NPAD, D), jnp.float32),
    ],
)
def _sc_hist(dstp_hbm, zeros_hbm, ones_hbm, out, dst_v, ones_v, hist_sh):
    c = lax.axis_index("c")
    s = lax.axis_index("s")
    w = c * NS + s
    pltpu.sync_copy(dstp_hbm.at[pl.ds(w * R_W, R_W)], dst_v)
    pltpu.sync_copy(ones_hbm, ones_v)
    pltpu.sync_copy(zeros_hbm, hist_sh.at[pl.ds(s * STRIPE, STRIPE)])
    plsc.subcore_barrier()

    def body(j, carry):
        pltpu.sync_copy(ones_v, hist_sh.at[dst_v.at[j]], add=True)
        return carry

    lax.fori_loop(0, R_W, body, 0)
    plsc.subcore_barrier()
    pltpu.sync_copy(hist_sh.at[pl.ds(s * STRIPE, STRIPE)],
                    out.at[c, pl.ds(s * STRIPE, STRIPE)])


@functools.partial(
    pl.kernel,
    out_type=jax.ShapeDtypeStruct((NC, NPAD, D), jnp.float32),
    mesh=_mesh,
    scratch_types=[
        pltpu.VMEM((R_W // 2, CHUNK), jnp.int32),
        pltpu.VMEM((R_W // 2, CHUNK), jnp.int32),
        pltpu.VMEM((CHUNK, D), jnp.float32),
        pltpu.VMEM((CHUNK, D), jnp.float32),
        pltpu.VMEM_SHARED((NPAD, D), jnp.float32),
        pltpu.SemaphoreType.DMA,
        pltpu.SemaphoreType.DMA,
    ],
)
def _sc_scatter(g_hbm, srcp_hbm, dstp_hbm, zeros_hbm, out,
                src_v, dst_v, b0, b1, acc_sh, s0, s1):
    c = lax.axis_index("c")
    s = lax.axis_index("s")
    w = c * NS + s
    bufs = (b0, b1)
    sems = (s0, s1)
    R_P = R_W // 2
    pltpu.sync_copy(zeros_hbm, acc_sh.at[pl.ds(s * STRIPE, STRIPE)])
    plsc.subcore_barrier()

    for phase in range(2):
        base = w * R_W + phase * R_P
        pltpu.sync_copy(srcp_hbm.at[pl.ds(base, R_P)], src_v)
        pltpu.sync_copy(dstp_hbm.at[pl.ds(base, R_P)], dst_v)
        pltpu.async_copy(g_hbm.at[src_v.at[0]], b0, s0)

        def body(t, carry):
            for b in range(2):
                j = 2 * t + b
                pltpu.make_async_copy(g_hbm.at[src_v.at[j]], bufs[b],
                                      sems[b]).wait()

                @pl.when(j + 1 < R_P)
                def _(b=b, j=j):
                    nb = (b + 1) % 2
                    pltpu.async_copy(g_hbm.at[src_v.at[j + 1]], bufs[nb],
                                     sems[nb])

                pltpu.sync_copy(bufs[b], acc_sh.at[dst_v.at[j]], add=True)
            return carry

        lax.fori_loop(0, R_P // 2, body, 0)
    plsc.subcore_barrier()
    pltpu.sync_copy(acc_sh.at[pl.ds(s * STRIPE, STRIPE)],
                    out.at[c, pl.ds(s * STRIPE, STRIPE)])


def _mm_body(x_ref, w_ref, o_ref):
    o_ref[...] = jnp.dot(x_ref[...], w_ref[...],
                         preferred_element_type=jnp.float32)


def _tc_matmul(xp, W):
    return pl.pallas_call(
        _mm_body,
        grid=(NPAD // BR,),
        in_specs=[pl.BlockSpec((BR, D), lambda i: (i, 0)),
                  pl.BlockSpec((D, D), lambda i: (0, 0))],
        out_specs=pl.BlockSpec((BR, D), lambda i: (i, 0)),
        out_shape=jax.ShapeDtypeStruct((NPAD, D), jnp.float32),
    )(xp, W)


_HSPEC0 = pl.BlockSpec((1, BR, D), lambda i: (0, i, 0))
_HSPEC1 = pl.BlockSpec((1, BR, D), lambda i: (1, i, 0))
_PSPEC0 = pl.BlockSpec((1, BR, D), lambda i: (0, i, 0))
_PSPEC1 = pl.BlockSpec((1, BR, D), lambda i: (1, i, 0))


def _dinv_of(h0_ref, h1_ref):
    deg = h0_ref[0] + h1_ref[0] + 1.0
    return lax.rsqrt(deg)


def _scale_body(h_ref, h0_ref, h1_ref, o_ref):
    o_ref[...] = h_ref[...] * _dinv_of(h0_ref, h1_ref)


def _tc_scale(H, hp):
    return pl.pallas_call(
        _scale_body,
        grid=(NPAD // BR,),
        in_specs=[pl.BlockSpec((BR, D), lambda i: (i, 0)), _HSPEC0, _HSPEC1],
        out_specs=pl.BlockSpec((BR, D), lambda i: (i, 0)),
        out_shape=jax.ShapeDtypeStruct((NPAD, D), jnp.float32),
    )(H, hp, hp)


def _layer_body(p_ref, q_ref, g_ref, h0_ref, h1_ref, b_ref, w_ref, o_ref):
    dinv = _dinv_of(h0_ref, h1_ref)
    hmid = jnp.maximum(
        dinv * (p_ref[0] + q_ref[0] + g_ref[...]) + b_ref[...], 0.0)
    o_ref[...] = dinv * jnp.dot(hmid, w_ref[...],
                                preferred_element_type=jnp.float32)


def _tc_layer(p, g, hp, b, W):
    return pl.pallas_call(
        _layer_body,
        grid=(NPAD // BR,),
        in_specs=[_PSPEC0, _PSPEC1,
                  pl.BlockSpec((BR, D), lambda i: (i, 0)),
                  _HSPEC0, _HSPEC1,
                  pl.BlockSpec((1, D), lambda i: (0, 0)),
                  pl.BlockSpec((D, D), lambda i: (0, 0))],
        out_specs=pl.BlockSpec((BR, D), lambda i: (i, 0)),
        out_shape=jax.ShapeDtypeStruct((NPAD, D), jnp.float32),
    )(p, p, g, hp, hp, b, W)


def _final_body(p_ref, q_ref, g_ref, h0_ref, h1_ref, b_ref, o_ref):
    dinv = _dinv_of(h0_ref, h1_ref)
    o_ref[...] = dinv * (p_ref[0] + q_ref[0] + g_ref[...]) + b_ref[...]


def _tc_final(p, g, hp, b):
    return pl.pallas_call(
        _final_body,
        grid=(NPAD // BR,),
        in_specs=[_PSPEC0, _PSPEC1,
                  pl.BlockSpec((BR, D), lambda i: (i, 0)),
                  _HSPEC0, _HSPEC1,
                  pl.BlockSpec((1, D), lambda i: (0, 0))],
        out_specs=pl.BlockSpec((BR, D), lambda i: (i, 0)),
        out_shape=jax.ShapeDtypeStruct((NPAD, D), jnp.float32),
    )(p, p, g, hp, hp, b)


def kernel(x, edge_index, W1, b1, W2, b2):
    src = edge_index[0]
    dst = edge_index[1]
    e = src.shape[0]
    fill = jnp.full((R_EDGE * CHUNK - e,), N, dtype=jnp.int32)
    srcp = jnp.concatenate([src, fill]).reshape(R_EDGE, CHUNK)
    dstp = jnp.concatenate([dst, fill]).reshape(R_EDGE, CHUNK)
    xp = jnp.pad(x, ((0, NPAD - N), (0, 0)))
    zD = jnp.zeros((STRIPE, D), jnp.float32)
    onesD = jnp.ones((CHUNK, D), jnp.float32)

    hp = _sc_hist(dstp, zD, onesD)
    H = _tc_matmul(xp, W1)
    g1 = _tc_scale(H, hp)
    p = _sc_scatter(g1, srcp, dstp, zD)
    g2 = _tc_layer(p, g1, hp, b1.reshape(1, D), W2)
    q = _sc_scatter(g2, srcp, dstp, zD)
    outp = _tc_final(q, g2, hp, b2.reshape(1, D))
    return outp[:N]


# speedup vs baseline: 10.4145x; 1.0522x over previous
import functools

import jax
import jax.numpy as jnp
from jax import lax
from jax.experimental import pallas as pl
from jax.experimental.pallas import tpu as pltpu
from jax.experimental.pallas import tpu_sc as plsc

N = 10000
D = 128
NPAD = 10240
NC = 2
NS = 16
NW = NC * NS
CHUNK = 128
R_EDGE = 2560
R_W = R_EDGE // NW
STRIPE = NPAD // NS
BR = 512

_mesh = plsc.VectorSubcoreMesh(core_axis_name="c", subcore_axis_name="s")


@functools.partial(
    pl.kernel,
    out_type=jax.ShapeDtypeStruct((NC, NPAD, D), jnp.float32),
    mesh=_mesh,
    scratch_types=[
        pltpu.VMEM((R_W, CHUNK), jnp.int32),
        pltpu.VMEM((CHUNK, D), jnp.float32),
        pltpu.VMEM_SHARED((NPAD, D), jnp.float32),
    ],
)
def _sc_hist(dstp_hbm, zeros_hbm, ones_hbm, out, dst_v, ones_v, hist_sh):
    c = lax.axis_index("c")
    s = lax.axis_index("s")
    w = c * NS + s
    pltpu.sync_copy(dstp_hbm.at[pl.ds(w * R_W, R_W)], dst_v)
    pltpu.sync_copy(ones_hbm, ones_v)
    pltpu.sync_copy(zeros_hbm, hist_sh.at[pl.ds(s * STRIPE, STRIPE)])
    plsc.subcore_barrier()

    def body(j, carry):
        pltpu.sync_copy(ones_v, hist_sh.at[dst_v.at[j]], add=True)
        return carry

    lax.fori_loop(0, R_W, body, 0)
    plsc.subcore_barrier()
    pltpu.sync_copy(hist_sh.at[pl.ds(s * STRIPE, STRIPE)],
                    out.at[c, pl.ds(s * STRIPE, STRIPE)])


@functools.partial(
    pl.kernel,
    out_type=jax.ShapeDtypeStruct((NC, NPAD, D), jnp.float32),
    mesh=_mesh,
    scratch_types=[
        pltpu.VMEM((R_W // 2, CHUNK), jnp.int32),
        pltpu.VMEM((R_W // 2, CHUNK), jnp.int32),
        pltpu.VMEM((CHUNK, D), jnp.float32),
        pltpu.VMEM((CHUNK, D), jnp.float32),
        pltpu.VMEM_SHARED((NPAD, D), jnp.float32),
        pltpu.SemaphoreType.DMA,
        pltpu.SemaphoreType.DMA,
    ],
)
def _sc_scatter(g_hbm, srcp_hbm, dstp_hbm, zeros_hbm, out,
                src_v, dst_v, b0, b1, acc_sh, s0, s1):
    c = lax.axis_index("c")
    s = lax.axis_index("s")
    w = c * NS + s
    bufs = (b0, b1)
    sems = (s0, s1)
    R_P = R_W // 2
    pltpu.sync_copy(zeros_hbm, acc_sh.at[pl.ds(s * STRIPE, STRIPE)])
    plsc.subcore_barrier()

    for phase in range(2):
        base = w * R_W + phase * R_P
        pltpu.sync_copy(srcp_hbm.at[pl.ds(base, R_P)], src_v)
        pltpu.sync_copy(dstp_hbm.at[pl.ds(base, R_P)], dst_v)
        pltpu.async_copy(g_hbm.at[src_v.at[0]], b0, s0)

        def body(t, carry):
            for b in range(2):
                j = 2 * t + b

                @pl.when(j + 1 < R_P)
                def _(b=b, j=j):
                    nb = (b + 1) % 2
                    pltpu.async_copy(g_hbm.at[src_v.at[j + 1]], bufs[nb],
                                     sems[nb])

                pltpu.make_async_copy(g_hbm.at[src_v.at[j]], bufs[b],
                                      sems[b]).wait()
                pltpu.sync_copy(bufs[b], acc_sh.at[dst_v.at[j]], add=True)
            return carry

        lax.fori_loop(0, R_P // 2, body, 0)
    plsc.subcore_barrier()
    pltpu.sync_copy(acc_sh.at[pl.ds(s * STRIPE, STRIPE)],
                    out.at[c, pl.ds(s * STRIPE, STRIPE)])


def _mm_body(x_ref, w_ref, o_ref):
    o_ref[...] = jnp.dot(x_ref[...], w_ref[...],
                         preferred_element_type=jnp.float32)


def _tc_matmul(xp, W):
    return pl.pallas_call(
        _mm_body,
        grid=(NPAD // BR,),
        in_specs=[pl.BlockSpec((BR, D), lambda i: (i, 0)),
                  pl.BlockSpec((D, D), lambda i: (0, 0))],
        out_specs=pl.BlockSpec((BR, D), lambda i: (i, 0)),
        out_shape=jax.ShapeDtypeStruct((NPAD, D), jnp.float32),
    )(xp, W)


_HSPEC0 = pl.BlockSpec((1, BR, D), lambda i: (0, i, 0))
_HSPEC1 = pl.BlockSpec((1, BR, D), lambda i: (1, i, 0))
_PSPEC0 = pl.BlockSpec((1, BR, D), lambda i: (0, i, 0))
_PSPEC1 = pl.BlockSpec((1, BR, D), lambda i: (1, i, 0))


def _dinv_of(h0_ref, h1_ref):
    deg = h0_ref[0] + h1_ref[0] + 1.0
    return lax.rsqrt(deg)


def _scale_body(h_ref, h0_ref, h1_ref, o_ref):
    o_ref[...] = h_ref[...] * _dinv_of(h0_ref, h1_ref)


def _tc_scale(H, hp):
    return pl.pallas_call(
        _scale_body,
        grid=(NPAD // BR,),
        in_specs=[pl.BlockSpec((BR, D), lambda i: (i, 0)), _HSPEC0, _HSPEC1],
        out_specs=pl.BlockSpec((BR, D), lambda i: (i, 0)),
        out_shape=jax.ShapeDtypeStruct((NPAD, D), jnp.float32),
    )(H, hp, hp)


def _layer_body(p_ref, q_ref, g_ref, h0_ref, h1_ref, b_ref, w_ref, o_ref):
    dinv = _dinv_of(h0_ref, h1_ref)
    hmid = jnp.maximum(
        dinv * (p_ref[0] + q_ref[0] + g_ref[...]) + b_ref[...], 0.0)
    o_ref[...] = dinv * jnp.dot(hmid, w_ref[...],
                                preferred_element_type=jnp.float32)


def _tc_layer(p, g, hp, b, W):
    return pl.pallas_call(
        _layer_body,
        grid=(NPAD // BR,),
        in_specs=[_PSPEC0, _PSPEC1,
                  pl.BlockSpec((BR, D), lambda i: (i, 0)),
                  _HSPEC0, _HSPEC1,
                  pl.BlockSpec((1, D), lambda i: (0, 0)),
                  pl.BlockSpec((D, D), lambda i: (0, 0))],
        out_specs=pl.BlockSpec((BR, D), lambda i: (i, 0)),
        out_shape=jax.ShapeDtypeStruct((NPAD, D), jnp.float32),
    )(p, p, g, hp, hp, b, W)


def _final_body(p_ref, q_ref, g_ref, h0_ref, h1_ref, b_ref, o_ref):
    dinv = _dinv_of(h0_ref, h1_ref)
    o_ref[...] = dinv * (p_ref[0] + q_ref[0] + g_ref[...]) + b_ref[...]


def _tc_final(p, g, hp, b):
    return pl.pallas_call(
        _final_body,
        grid=(NPAD // BR,),
        in_specs=[_PSPEC0, _PSPEC1,
                  pl.BlockSpec((BR, D), lambda i: (i, 0)),
                  _HSPEC0, _HSPEC1,
                  pl.BlockSpec((1, D), lambda i: (0, 0))],
        out_specs=pl.BlockSpec((BR, D), lambda i: (i, 0)),
        out_shape=jax.ShapeDtypeStruct((NPAD, D), jnp.float32),
    )(p, p, g, hp, hp, b)


def kernel(x, edge_index, W1, b1, W2, b2):
    src = edge_index[0]
    dst = edge_index[1]
    e = src.shape[0]
    fill = jnp.full((R_EDGE * CHUNK - e,), N, dtype=jnp.int32)
    srcp = jnp.concatenate([src, fill]).reshape(R_EDGE, CHUNK)
    dstp = jnp.concatenate([dst, fill]).reshape(R_EDGE, CHUNK)
    xp = jnp.pad(x, ((0, NPAD - N), (0, 0)))
    zD = jnp.zeros((STRIPE, D), jnp.float32)
    onesD = jnp.ones((CHUNK, D), jnp.float32)

    hp = _sc_hist(dstp, zD, onesD)
    H = _tc_matmul(xp, W1)
    g1 = _tc_scale(H, hp)
    p = _sc_scatter(g1, srcp, dstp, zD)
    g2 = _tc_layer(p, g1, hp, b1.reshape(1, D), W2)
    q = _sc_scatter(g2, srcp, dstp, zD)
    outp = _tc_final(q, g2, hp, b2.reshape(1, D))
    return outp[:N]
